# baseline (speedup 1.0000x reference)
import jax
import jax.numpy as jnp
from jax import lax
from jax.experimental import pallas as pl
from jax.experimental.pallas import tpu as pltpu
from jax.experimental.pallas import tpu_sc as plsc

B, S, I = 16, 32, 1024
H = 1024
M = 1024
D = 256
O = 1024
TOPK = 3

NW = 32
QP = (S * B) // NW
L = 16


def _xw_body(x_ref, w_ref, b_ref, o_ref):
    o_ref[...] = (
        jax.lax.dot_general(
            x_ref[...], w_ref[...], (((1,), (1,)), ((), ())),
            preferred_element_type=jnp.float32,
        )
        + b_ref[...]
    )


def _compute_xw(x_sb, Wih, bsum):
    NBLK = 8
    blk = (4 * H) // NBLK
    return pl.pallas_call(
        _xw_body,
        grid=(NBLK,),
        in_specs=[
            pl.BlockSpec((S * B, I), lambda n: (0, 0)),
            pl.BlockSpec((blk, I), lambda n: (n, 0)),
            pl.BlockSpec((1, blk), lambda n: (0, n)),
        ],
        out_specs=pl.BlockSpec((S * B, blk), lambda n: (0, n)),
        out_shape=jax.ShapeDtypeStruct((S * B, 4 * H), jnp.float32),
    )(x_sb, Wih, bsum)


def _scan_body(xw_ref, whh_ref, wq_ref, bq_ref, kmem_ref, woh_ref, bo_ref,
               simsT3_ref, partial_ref, co_ref):
    C = 256
    NCK = H // C

    c_parts = []
    for ck in range(NCK):
        xi = xw_ref[0:B, 0 * H + ck * C:0 * H + (ck + 1) * C]
        xg = xw_ref[0:B, 2 * H + ck * C:2 * H + (ck + 1) * C]
        xo = xw_ref[0:B, 3 * H + ck * C:3 * H + (ck + 1) * C]
        c_ck = jax.nn.sigmoid(xi) * jnp.tanh(xg)
        co_ref[0:B, ck * C:(ck + 1) * C] = jax.nn.sigmoid(xo) * jnp.tanh(c_ck)
        c_parts.append(c_ck)

    for t in range(1, S):
        hp = co_ref[(t - 1) * B:t * B, :]
        new_parts = []
        for ck in range(NCK):
            def gate(g):
                w = whh_ref[g * H + ck * C:g * H + (ck + 1) * C, :]
                return xw_ref[t * B:(t + 1) * B, g * H + ck * C:g * H + (ck + 1) * C] + \
                    jax.lax.dot_general(hp, w, (((1,), (1,)), ((), ())),
                                        preferred_element_type=jnp.float32)
            c_ck = (jax.nn.sigmoid(gate(1)) * c_parts[ck]
                    + jax.nn.sigmoid(gate(0)) * jnp.tanh(gate(2)))
            co_ref[t * B:(t + 1) * B, ck * C:(ck + 1) * C] = \
                jax.nn.sigmoid(gate(3)) * jnp.tanh(c_ck)
            new_parts.append(c_ck)
        c_parts = new_parts

    co = co_ref[...]
    q = jax.lax.dot_general(co, wq_ref[...], (((1,), (1,)), ((), ())),
                            preferred_element_type=jnp.float32) + bq_ref[...]
    qn = q / jnp.maximum(jnp.sqrt(jnp.sum(q * q, axis=1, keepdims=True)), 1e-12)
    km = kmem_ref[...]
    kn = km / jnp.maximum(jnp.sqrt(jnp.sum(km * km, axis=1, keepdims=True)), 1e-12)
    simsT = jax.lax.dot_general(kn, qn, (((1,), (1,)), ((), ())),
                                preferred_element_type=jnp.float32)
    for w in range(NW):
        simsT3_ref[w] = simsT[:, w * QP:(w + 1) * QP]
    partial_ref[...] = (
        jax.lax.dot_general(co, woh_ref[...], (((1,), (1,)), ((), ())),
                            preferred_element_type=jnp.float32)
        + bo_ref[...]
    )


def _run_scan(xw, Whh, Wq, bq2, Kmem, WoH, bo2):
    return pl.pallas_call(
        _scan_body,
        out_shape=(
            jax.ShapeDtypeStruct((NW, M, QP), jnp.float32),
            jax.ShapeDtypeStruct((S * B, O), jnp.float32),
        ),
        scratch_shapes=[pltpu.VMEM((S * B, H), jnp.float32)],
    )(xw, Whh, Wq, bq2, Kmem, WoH, bo2)


def _retrieve_body(simsT3_hbm, vmem_hbm, rows_hbm, idx_hbm,
                   st_v, idx_v, rows_v, sem):
    wid = lax.axis_index("s") * 2 + lax.axis_index("c")
    base = wid * QP
    pltpu.sync_copy(simsT3_hbm.at[wid], st_v)

    neg = jnp.full((L,), -jnp.inf, jnp.float32)
    zero_i = jnp.zeros((L,), jnp.int32)

    UNROLL = 4

    def body(mm, carry):
        for u in range(UNROLL):
            m = mm * UNROLL + u
            t1v, t1i, t2v, t2i, t3v, t3i = carry
            v = st_v[m]
            vi = jnp.full((L,), m, jnp.int32)
            gt1 = v > t1v
            nt1v = jnp.where(gt1, v, t1v)
            nt1i = jnp.where(gt1, vi, t1i)
            d1v = jnp.where(gt1, t1v, v)
            d1i = jnp.where(gt1, t1i, vi)
            gt2 = d1v > t2v
            nt2v = jnp.where(gt2, d1v, t2v)
            nt2i = jnp.where(gt2, d1i, t2i)
            d2v = jnp.where(gt2, t2v, d1v)
            d2i = jnp.where(gt2, t2i, d1i)
            gt3 = d2v > t3v
            nt3v = jnp.where(gt3, d2v, t3v)
            nt3i = jnp.where(gt3, d2i, t3i)
            carry = (nt1v, nt1i, nt2v, nt2i, nt3v, nt3i)
        return carry

    init = (neg, zero_i, neg, zero_i, neg, zero_i)
    t1v, t1i, t2v, t2i, t3v, t3i = lax.fori_loop(0, M // UNROLL, body, init)

    idx_v[0 * QP:1 * QP] = t1i
    idx_v[1 * QP:2 * QP] = t2i
    idx_v[2 * QP:3 * QP] = t3i
    pltpu.async_copy(vmem_hbm.at[idx_v], rows_v, sem).wait()

    for k in range(TOPK):
        pltpu.sync_copy(rows_v.at[pl.ds(k * QP, QP)],
                        rows_hbm.at[pl.ds(k * S * B + base, QP)])
        pltpu.sync_copy(idx_v.at[pl.ds(k * QP, QP)],
                        idx_hbm.at[pl.ds(k * S * B + base, QP)])


def _run_retrieve(simsT3, Vmem):
    mesh = plsc.VectorSubcoreMesh(core_axis_name="c", subcore_axis_name="s")
    k = pl.kernel(
        _retrieve_body,
        mesh=mesh,
        compiler_params=pltpu.CompilerParams(use_tc_tiling_on_sc=False),
        out_type=(
            jax.ShapeDtypeStruct((TOPK * S * B, D), jnp.float32),
            jax.ShapeDtypeStruct((TOPK * S * B,), jnp.int32),
        ),
        scratch_types=[
            pltpu.VMEM((M, QP), jnp.float32),
            pltpu.VMEM((TOPK * QP,), jnp.int32),
            pltpu.VMEM((TOPK * QP, D), jnp.float32),
            pltpu.SemaphoreType.DMA,
        ],
    )
    return k(simsT3, Vmem)


def _combine_body(partial_ref, rows_ref, idx_ref, vmem_ref, wa_ref,
                  wc_ref, bc_ref, wod_ref, o_ref):
    vl = jax.lax.dot_general(wa_ref[...], vmem_ref[...], (((1,), (1,)), ((), ())),
                             preferred_element_type=jnp.float32)
    lane = jax.lax.broadcasted_iota(jnp.int32, (S * B, M), 1)
    logits = []
    for k in range(TOPK):
        idx_k = idx_ref[k * S * B:(k + 1) * S * B, 0:1]
        oh = (lane == idx_k).astype(jnp.float32)
        logits.append(jnp.sum(oh * vl, axis=1, keepdims=True))
    lmax = jnp.maximum(jnp.maximum(logits[0], logits[1]), logits[2])
    e = [jnp.exp(lg - lmax) for lg in logits]
    es = e[0] + e[1] + e[2]
    mem = (e[0] * rows_ref[0 * S * B:1 * S * B, :]
           + e[1] * rows_ref[1 * S * B:2 * S * B, :]
           + e[2] * rows_ref[2 * S * B:3 * S * B, :]) / es
    memc = jax.lax.dot_general(mem, wc_ref[...], (((1,), (1,)), ((), ())),
                               preferred_element_type=jnp.float32) + bc_ref[...]
    o_ref[...] = partial_ref[...] + jax.lax.dot_general(
        memc, wod_ref[...], (((1,), (1,)), ((), ())),
        preferred_element_type=jnp.float32)


def _run_combine(partial, rows, idx, Vmem, Wa, Wc, bc2, WoD):
    return pl.pallas_call(
        _combine_body,
        out_shape=jax.ShapeDtypeStruct((S * B, O), jnp.float32),
    )(partial, rows, idx.reshape(TOPK * S * B, 1), Vmem, Wa, Wc, bc2, WoD)


def kernel(x, Wih, Whh, bih, bhh, Wq, bq, Wa, ba, Wc, bc, Wo, bo, Kmem, Vmem):
    x_sb = jnp.transpose(x, (1, 0, 2)).reshape(S * B, I)
    bsum = (bih + bhh).reshape(1, 4 * H)
    xw = _compute_xw(x_sb, Wih, bsum)
    simsT3, partial = _run_scan(xw, Whh, Wq, bq.reshape(1, D), Kmem,
                                Wo[:, :H], bo.reshape(1, O))
    rows, idx = _run_retrieve(simsT3, Vmem)
    out_flat = _run_combine(partial, rows, idx, Vmem, Wa, Wc,
                            bc.reshape(1, D), Wo[:, H:])
    return jnp.transpose(out_flat.reshape(S, B, O), (1, 0, 2))


# speedup vs baseline: 1.1231x; 1.1231x over previous
import jax
import jax.numpy as jnp
from jax import lax
from jax.experimental import pallas as pl
from jax.experimental.pallas import tpu as pltpu
from jax.experimental.pallas import tpu_sc as plsc

B, S, I = 16, 32, 1024
H = 1024
M = 1024
D = 256
O = 1024
TOPK = 3

NW = 32
QP = (S * B) // NW
L = 16


def _xw_body(x_ref, w_ref, b_ref, o_ref):
    o_ref[...] = (
        jax.lax.dot_general(
            x_ref[...], w_ref[...], (((1,), (1,)), ((), ())),
            preferred_element_type=jnp.float32,
        )
        + b_ref[...]
    )


def _compute_xw(x_sb, Wih, bsum):
    NBLK = 8
    blk = (4 * H) // NBLK
    return pl.pallas_call(
        _xw_body,
        grid=(NBLK,),
        in_specs=[
            pl.BlockSpec((S * B, I), lambda n: (0, 0)),
            pl.BlockSpec((blk, I), lambda n: (n, 0)),
            pl.BlockSpec((1, blk), lambda n: (0, n)),
        ],
        out_specs=pl.BlockSpec((S * B, blk), lambda n: (0, n)),
        out_shape=jax.ShapeDtypeStruct((S * B, 4 * H), jnp.float32),
    )(x_sb, Wih, bsum)


def _scan_body(xw_ref, whh_ref, wq_ref, bq_ref, kmem_ref, woh_ref, bo_ref,
               simsT3_ref, partial_ref, co_ref):
    C = 256
    NCK = H // C

    c_parts = []
    for ck in range(NCK):
        xi = xw_ref[0:B, 0 * H + ck * C:0 * H + (ck + 1) * C]
        xg = xw_ref[0:B, 2 * H + ck * C:2 * H + (ck + 1) * C]
        xo = xw_ref[0:B, 3 * H + ck * C:3 * H + (ck + 1) * C]
        c_ck = jax.nn.sigmoid(xi) * jnp.tanh(xg)
        co_ref[0:B, ck * C:(ck + 1) * C] = jax.nn.sigmoid(xo) * jnp.tanh(c_ck)
        c_parts.append(c_ck)

    for t in range(1, S):
        hp = co_ref[(t - 1) * B:t * B, :]
        new_parts = []
        for ck in range(NCK):
            def gate(g):
                w = whh_ref[g * H + ck * C:g * H + (ck + 1) * C, :]
                return xw_ref[t * B:(t + 1) * B, g * H + ck * C:g * H + (ck + 1) * C] + \
                    jax.lax.dot_general(hp, w, (((1,), (1,)), ((), ())),
                                        preferred_element_type=jnp.float32)
            c_ck = (jax.nn.sigmoid(gate(1)) * c_parts[ck]
                    + jax.nn.sigmoid(gate(0)) * jnp.tanh(gate(2)))
            co_ref[t * B:(t + 1) * B, ck * C:(ck + 1) * C] = \
                jax.nn.sigmoid(gate(3)) * jnp.tanh(c_ck)
            new_parts.append(c_ck)
        c_parts = new_parts

    co = co_ref[...]
    q = jax.lax.dot_general(co, wq_ref[...], (((1,), (1,)), ((), ())),
                            preferred_element_type=jnp.float32) + bq_ref[...]
    qn = q / jnp.maximum(jnp.sqrt(jnp.sum(q * q, axis=1, keepdims=True)), 1e-12)
    km = kmem_ref[...]
    kn = km / jnp.maximum(jnp.sqrt(jnp.sum(km * km, axis=1, keepdims=True)), 1e-12)
    simsT = jax.lax.dot_general(kn, qn, (((1,), (1,)), ((), ())),
                                preferred_element_type=jnp.float32)
    for g in range(NW // 8):
        simsT3_ref[g] = simsT[:, g * 8 * QP:(g + 1) * 8 * QP]
    partial_ref[...] = (
        jax.lax.dot_general(co, woh_ref[...], (((1,), (1,)), ((), ())),
                            preferred_element_type=jnp.float32)
        + bo_ref[...]
    )


def _run_scan(xw, Whh, Wq, bq2, Kmem, WoH, bo2):
    return pl.pallas_call(
        _scan_body,
        out_shape=(
            jax.ShapeDtypeStruct((NW // 8, M, 8 * QP), jnp.float32),
            jax.ShapeDtypeStruct((S * B, O), jnp.float32),
        ),
        scratch_shapes=[pltpu.VMEM((S * B, H), jnp.float32)],
    )(xw, Whh, Wq, bq2, Kmem, WoH, bo2)


def _retrieve_body(simsT3_hbm, vmem_hbm, rows_hbm, idx_hbm,
                   st_v, idx_v, rows_v, sem):
    wid = lax.axis_index("s") * 2 + lax.axis_index("c")
    base = wid * QP
    pltpu.sync_copy(simsT3_hbm.at[wid // 8, :, pl.ds((wid % 8) * QP, QP)], st_v)

    neg = jnp.full((L,), -jnp.inf, jnp.float32)
    zero_i = jnp.zeros((L,), jnp.int32)

    UNROLL = 4

    def body(mm, carry):
        for u in range(UNROLL):
            m = mm * UNROLL + u
            t1v, t1i, t2v, t2i, t3v, t3i = carry
            v = st_v[m]
            vi = jnp.full((L,), m, jnp.int32)
            gt1 = v > t1v
            nt1v = jnp.where(gt1, v, t1v)
            nt1i = jnp.where(gt1, vi, t1i)
            d1v = jnp.where(gt1, t1v, v)
            d1i = jnp.where(gt1, t1i, vi)
            gt2 = d1v > t2v
            nt2v = jnp.where(gt2, d1v, t2v)
            nt2i = jnp.where(gt2, d1i, t2i)
            d2v = jnp.where(gt2, t2v, d1v)
            d2i = jnp.where(gt2, t2i, d1i)
            gt3 = d2v > t3v
            nt3v = jnp.where(gt3, d2v, t3v)
            nt3i = jnp.where(gt3, d2i, t3i)
            carry = (nt1v, nt1i, nt2v, nt2i, nt3v, nt3i)
        return carry

    init = (neg, zero_i, neg, zero_i, neg, zero_i)
    t1v, t1i, t2v, t2i, t3v, t3i = lax.fori_loop(0, M // UNROLL, body, init)

    idx_v[0 * QP:1 * QP] = t1i
    idx_v[1 * QP:2 * QP] = t2i
    idx_v[2 * QP:3 * QP] = t3i
    pltpu.async_copy(vmem_hbm.at[idx_v], rows_v, sem).wait()

    for k in range(TOPK):
        pltpu.sync_copy(rows_v.at[pl.ds(k * QP, QP)],
                        rows_hbm.at[pl.ds(k * S * B + base, QP)])
        pltpu.sync_copy(idx_v.at[pl.ds(k * QP, QP)],
                        idx_hbm.at[pl.ds(k * S * B + base, QP)])


def _run_retrieve(simsT3, Vmem):
    mesh = plsc.VectorSubcoreMesh(core_axis_name="c", subcore_axis_name="s")
    k = pl.kernel(
        _retrieve_body,
        mesh=mesh,
        compiler_params=pltpu.CompilerParams(use_tc_tiling_on_sc=False),
        out_type=(
            jax.ShapeDtypeStruct((TOPK * S * B, D), jnp.float32),
            jax.ShapeDtypeStruct((TOPK * S * B,), jnp.int32),
        ),
        scratch_types=[
            pltpu.VMEM((M, QP), jnp.float32),
            pltpu.VMEM((TOPK * QP,), jnp.int32),
            pltpu.VMEM((TOPK * QP, D), jnp.float32),
            pltpu.SemaphoreType.DMA,
        ],
    )
    return k(simsT3, Vmem)


def _combine_body(partial_ref, rows_ref, idx_ref, vmem_ref, wa_ref,
                  wc_ref, bc_ref, wod_ref, o_ref):
    vl = jax.lax.dot_general(wa_ref[...], vmem_ref[...], (((1,), (1,)), ((), ())),
                             preferred_element_type=jnp.float32)
    lane = jax.lax.broadcasted_iota(jnp.int32, (S * B, M), 1)
    logits = []
    for k in range(TOPK):
        idx_k = idx_ref[k * S * B:(k + 1) * S * B, 0:1]
        oh = (lane == idx_k).astype(jnp.float32)
        logits.append(jnp.sum(oh * vl, axis=1, keepdims=True))
    lmax = jnp.maximum(jnp.maximum(logits[0], logits[1]), logits[2])
    e = [jnp.exp(lg - lmax) for lg in logits]
    es = e[0] + e[1] + e[2]
    mem = (e[0] * rows_ref[0 * S * B:1 * S * B, :]
           + e[1] * rows_ref[1 * S * B:2 * S * B, :]
           + e[2] * rows_ref[2 * S * B:3 * S * B, :]) / es
    memc = jax.lax.dot_general(mem, wc_ref[...], (((1,), (1,)), ((), ())),
                               preferred_element_type=jnp.float32) + bc_ref[...]
    o_ref[...] = partial_ref[...] + jax.lax.dot_general(
        memc, wod_ref[...], (((1,), (1,)), ((), ())),
        preferred_element_type=jnp.float32)


def _run_combine(partial, rows, idx, Vmem, Wa, Wc, bc2, WoD):
    return pl.pallas_call(
        _combine_body,
        out_shape=jax.ShapeDtypeStruct((S * B, O), jnp.float32),
    )(partial, rows, idx.reshape(TOPK * S * B, 1), Vmem, Wa, Wc, bc2, WoD)


def kernel(x, Wih, Whh, bih, bhh, Wq, bq, Wa, ba, Wc, bc, Wo, bo, Kmem, Vmem):
    x_sb = jnp.transpose(x, (1, 0, 2)).reshape(S * B, I)
    bsum = (bih + bhh).reshape(1, 4 * H)
    xw = _compute_xw(x_sb, Wih, bsum)
    simsT3, partial = _run_scan(xw, Whh, Wq, bq.reshape(1, D), Kmem,
                                Wo[:, :H], bo.reshape(1, O))
    rows, idx = _run_retrieve(simsT3, Vmem)
    out_flat = _run_combine(partial, rows, idx, Vmem, Wa, Wc,
                            bc.reshape(1, D), Wo[:, H:])
    return jnp.transpose(out_flat.reshape(S, B, O), (1, 0, 2))
